# Initial kernel scaffold; baseline (speedup 1.0000x reference)
import jax
import jax.numpy as jnp
from jax.experimental import pallas as pl


def kernel(features):
    raise NotImplementedError("write your pallas kernel here")


# speedup vs baseline: 1.0002x; 1.0002x over previous
import jax
import jax.numpy as jnp
from jax.experimental import pallas as pl
from jax.experimental.pallas import tpu as pltpu

B, PH, PW, D = 8, 16, 16, 1024
P = PH * PW


def _prep_kernel(f_ref, out_ref):
    x = f_ref[0]
    mu = jnp.mean(x, axis=-1, keepdims=True)
    var = jnp.mean((x - mu) * (x - mu), axis=-1, keepdims=True)
    xn = (x - mu) / jnp.sqrt(var + 1e-6)
    out_ref[0, 0] = xn

    def pool(a):
        zv = jnp.zeros((1, PW, D), dtype=a.dtype)
        v = a + jnp.concatenate([a[1:], zv], axis=0) + jnp.concatenate([zv, a[:-1]], axis=0)
        zh = jnp.zeros((PH, 1, D), dtype=a.dtype)
        return v + jnp.concatenate([v[:, 1:], zh], axis=1) + jnp.concatenate([zh, v[:, :-1]], axis=1)

    s = pool(xn)
    c = pool(jnp.ones_like(xn))
    out_ref[0, 1] = s / c


def _main_kernel(xq_ref, xk_ref, ps_ref, sc_ref, m1, m2, acc):
    i = pl.program_id(0)
    j = pl.program_id(2)
    r = pl.program_id(1)

    @pl.when(j == 0)
    def _():
        m1[...] = jnp.full((1, P), jnp.inf, jnp.float32)
        m2[...] = jnp.full((1, P), jnp.inf, jnp.float32)

    @pl.when(j != i)
    def _():
        xq = xq_ref[0, 0]
        xk = xk_ref[0, 0]
        sqk = jnp.sum(xk * xk, axis=1, keepdims=True)
        ones_row = jnp.ones((1, D), jnp.float32)
        sqq = jax.lax.dot_general(ones_row, xq * xq,
                                  (((1,), (1,)), ((), ())),
                                  preferred_element_type=jnp.float32)
        g = jax.lax.dot_general(xk, xq, (((1,), (1,)), ((), ())),
                                preferred_element_type=jnp.float32)
        d = jnp.min(sqk - 2.0 * g, axis=0, keepdims=True) + sqq
        old1 = m1[...]
        m1[...] = jnp.minimum(old1, d)
        m2[...] = jnp.where(d < old1, old1, jnp.minimum(m2[...], d))

    @pl.when(j == B - 1)
    def _():
        contrib = 0.5 * (jnp.sqrt(jnp.maximum(m1[...], 1e-12)) +
                         jnp.sqrt(jnp.maximum(m2[...], 1e-12)))

        @pl.when(r == 0)
        def _():
            acc[...] = 0.5 * contrib

        @pl.when(r == 1)
        def _():
            tot = acc[...] + 0.5 * contrib
            ps_ref[0] = tot
            sc_ref[0] = jnp.broadcast_to(jnp.max(tot, axis=1, keepdims=True), (1, 128))


def kernel(features):
    f4 = features.reshape(B, PH, PW, D)
    xs = pl.pallas_call(
        _prep_kernel,
        grid=(B,),
        in_specs=[pl.BlockSpec((1, PH, PW, D), lambda b: (b, 0, 0, 0))],
        out_specs=pl.BlockSpec((1, 2, PH, PW, D), lambda b: (b, 0, 0, 0, 0)),
        out_shape=jax.ShapeDtypeStruct((B, 2, PH, PW, D), jnp.float32),
    )(f4)
    xs = xs.reshape(B, 2, P, D)

    ps, sc = pl.pallas_call(
        _main_kernel,
        grid=(B, 2, B),
        in_specs=[
            pl.BlockSpec((1, 1, P, D), lambda i, r, j: (i, r, 0, 0)),
            pl.BlockSpec((1, 1, P, D), lambda i, r, j: (j, r, 0, 0)),
        ],
        out_specs=[
            pl.BlockSpec((1, 1, P), lambda i, r, j: (i, 0, 0)),
            pl.BlockSpec((1, 1, 128), lambda i, r, j: (i, 0, 0)),
        ],
        out_shape=[
            jax.ShapeDtypeStruct((B, 1, P), jnp.float32),
            jax.ShapeDtypeStruct((B, 1, 128), jnp.float32),
        ],
        scratch_shapes=[
            pltpu.VMEM((1, P), jnp.float32),
            pltpu.VMEM((1, P), jnp.float32),
            pltpu.VMEM((1, P), jnp.float32),
        ],
    )(xs, xs)

    scores = sc[:, 0, 0]
    scores_pixel = ps.reshape(B, PH, PW)
    return scores, scores_pixel


# speedup vs baseline: 2.7344x; 2.7339x over previous
import jax
import jax.numpy as jnp
import numpy as np
from jax.experimental import pallas as pl
from jax.experimental.pallas import tpu as pltpu

B, PH, PW, D = 8, 16, 16, 1024
P = PH * PW
N = B * P
NPAIR = B * (B - 1) // 2


def _prep(f_ref, xs):
    x = f_ref[...].reshape(N, D)
    mu = jnp.mean(x, axis=-1, keepdims=True)
    var = jnp.mean((x - mu) * (x - mu), axis=-1, keepdims=True)
    xn = (x - mu) / jnp.sqrt(var + 1e-6)
    xs[0] = xn

    io = jax.lax.broadcasted_iota(jnp.int32, (N, 1), 0)
    pm = io % P
    ph = pm // PW
    pw = pm % PW
    z16 = jnp.zeros((16, D), jnp.float32)
    z1 = jnp.zeros((1, D), jnp.float32)
    up = jnp.concatenate([xn[16:], z16], axis=0)
    dn = jnp.concatenate([z16, xn[:-16]], axis=0)
    vs = xn + jnp.where(ph < PH - 1, up, 0.0) + jnp.where(ph > 0, dn, 0.0)
    lf = jnp.concatenate([vs[1:], z1], axis=0)
    rt = jnp.concatenate([z1, vs[:-1]], axis=0)
    hs = vs + jnp.where(pw < PW - 1, lf, 0.0) + jnp.where(pw > 0, rt, 0.0)
    cv = 3.0 - (ph == 0) - (ph == PH - 1)
    ch = 3.0 - (pw == 0) - (pw == PW - 1)
    xs[1] = hs / (cv * ch)


def _merge_top2(a1, a2, b1, b2):
    m1 = jnp.minimum(a1, b1)
    m2 = jnp.minimum(jnp.maximum(a1, b1), jnp.minimum(a2, b2))
    return m1, m2


def _fused_kernel(ii_ref, jj_ref, f_ref, ps_ref, sc_ref,
                  xs, m1r, m2r, m1c, m2c, acc):
    r = pl.program_id(0)
    p = pl.program_id(1)

    @pl.when((r == 0) & (p == 0))
    def _():
        _prep(f_ref, xs)

    @pl.when(p == 0)
    def _():
        m1r[...] = jnp.full((B, P), jnp.inf, jnp.float32)
        m2r[...] = jnp.full((B, P), jnp.inf, jnp.float32)
        m1c[...] = jnp.full((P, B), jnp.inf, jnp.float32)
        m2c[...] = jnp.full((P, B), jnp.inf, jnp.float32)

    i = ii_ref[p]
    j = jj_ref[p]
    xq = xs[r, pl.ds(i * P, P)]
    xk = xs[r, pl.ds(j * P, P)]
    sqk = jnp.sum(xk * xk, axis=1, keepdims=True)
    ones_row = jnp.ones((1, D), jnp.float32)
    sqq = jax.lax.dot_general(ones_row, xq * xq,
                              (((1,), (1,)), ((), ())),
                              preferred_element_type=jnp.float32)
    g = jax.lax.dot_general(xk, xq, (((1,), (1,)), ((), ())),
                            preferred_element_type=jnp.float32)
    d2 = (sqk - 2.0 * g) + sqq
    dq = jnp.min(d2, axis=0, keepdims=True)
    dk = jnp.min(d2, axis=1, keepdims=True)

    rows = jax.lax.broadcasted_iota(jnp.int32, (B, P), 0)
    urow = rows == i
    dqb = jnp.broadcast_to(dq, (B, P))
    o1 = m1r[...]
    m1r[...] = jnp.where(urow, jnp.minimum(o1, dqb), o1)
    m2r[...] = jnp.where(urow & (dqb < o1), o1,
                         jnp.where(urow, jnp.minimum(m2r[...], dqb), m2r[...]))

    cols = jax.lax.broadcasted_iota(jnp.int32, (P, B), 1)
    ucol = cols == j
    dkb = jnp.broadcast_to(dk, (P, B))
    c1 = m1c[...]
    m1c[...] = jnp.where(ucol, jnp.minimum(c1, dkb), c1)
    m2c[...] = jnp.where(ucol & (dkb < c1), c1,
                         jnp.where(ucol, jnp.minimum(m2c[...], dkb), m2c[...]))

    @pl.when(p == NPAIR - 1)
    def _():
        t1 = m1c[...].T
        t2 = m2c[...].T
        f1, f2 = _merge_top2(m1r[...], m2r[...], t1, t2)
        contrib = 0.5 * (jnp.sqrt(jnp.maximum(f1, 1e-12)) +
                         jnp.sqrt(jnp.maximum(f2, 1e-12)))

        @pl.when(r == 0)
        def _():
            acc[...] = 0.5 * contrib

        @pl.when(r == 1)
        def _():
            tot = acc[...] + 0.5 * contrib
            ps_ref[...] = tot
            sc_ref[...] = jnp.broadcast_to(jnp.max(tot, axis=1, keepdims=True),
                                           (B, 128))


def kernel(features):
    pairs = [(a, b) for a in range(B) for b in range(a + 1, B)]
    ii = jnp.asarray(np.array([a for a, _ in pairs], dtype=np.int32))
    jj = jnp.asarray(np.array([b for _, b in pairs], dtype=np.int32))

    ps, sc = pl.pallas_call(
        _fused_kernel,
        grid_spec=pltpu.PrefetchScalarGridSpec(
            num_scalar_prefetch=2,
            grid=(2, NPAIR),
            in_specs=[pl.BlockSpec((B, P, D), lambda r, p, ii, jj: (0, 0, 0))],
            out_specs=[
                pl.BlockSpec((B, P), lambda r, p, ii, jj: (0, 0)),
                pl.BlockSpec((B, 128), lambda r, p, ii, jj: (0, 0)),
            ],
            scratch_shapes=[
                pltpu.VMEM((2, N, D), jnp.float32),
                pltpu.VMEM((B, P), jnp.float32),
                pltpu.VMEM((B, P), jnp.float32),
                pltpu.VMEM((P, B), jnp.float32),
                pltpu.VMEM((P, B), jnp.float32),
                pltpu.VMEM((B, P), jnp.float32),
            ],
        ),
        out_shape=[
            jax.ShapeDtypeStruct((B, P), jnp.float32),
            jax.ShapeDtypeStruct((B, 128), jnp.float32),
        ],
    )(ii, jj, features)

    scores = sc[:, 0]
    scores_pixel = ps.reshape(B, PH, PW)
    return scores, scores_pixel


# speedup vs baseline: 2.8729x; 1.0507x over previous
import jax
import jax.numpy as jnp
import numpy as np
from jax.experimental import pallas as pl
from jax.experimental.pallas import tpu as pltpu

B, PH, PW, D = 8, 16, 16, 1024
P = PH * PW
N = B * P
NPAIR = B * (B - 1) // 2


def _prep(f_ref, xs, sqc, sqr):
    x = f_ref[...].reshape(N, D)
    mu = jnp.mean(x, axis=-1, keepdims=True)
    var = jnp.mean((x - mu) * (x - mu), axis=-1, keepdims=True)
    xn = (x - mu) / jnp.sqrt(var + 1e-6)

    io = jax.lax.broadcasted_iota(jnp.int32, (N, 1), 0)
    pm = io % P
    ph = pm // PW
    pw = pm % PW
    z16 = jnp.zeros((16, D), jnp.float32)
    z1 = jnp.zeros((1, D), jnp.float32)
    up = jnp.concatenate([xn[16:], z16], axis=0)
    dn = jnp.concatenate([z16, xn[:-16]], axis=0)
    vs = xn + jnp.where(ph < PH - 1, up, 0.0) + jnp.where(ph > 0, dn, 0.0)
    lf = jnp.concatenate([vs[1:], z1], axis=0)
    rt = jnp.concatenate([z1, vs[:-1]], axis=0)
    hs = vs + jnp.where(pw < PW - 1, lf, 0.0) + jnp.where(pw > 0, rt, 0.0)
    cv = 3.0 - (ph == 0) - (ph == PH - 1)
    ch = 3.0 - (pw == 0) - (pw == PW - 1)
    pooled = hs / (cv * ch)

    xs[0] = xn.astype(jnp.bfloat16)
    xs[1] = pooled.astype(jnp.bfloat16)
    ones_row = jnp.ones((1, D), jnp.float32)
    for r in range(2):
        a = [xn, pooled][r]
        sqc[r] = jnp.sum(a * a, axis=1, keepdims=True)
        for b in range(B):
            zb = a[b * P:(b + 1) * P]
            sqr[r, pl.ds(b, 1)] = jax.lax.dot_general(
                ones_row, zb * zb, (((1,), (1,)), ((), ())),
                preferred_element_type=jnp.float32)


def _merge_top2(a1, a2, b1, b2):
    m1 = jnp.minimum(a1, b1)
    m2 = jnp.minimum(jnp.maximum(a1, b1), jnp.minimum(a2, b2))
    return m1, m2


def _fused_kernel(ii_ref, jj_ref, f_ref, ps_ref, sc_ref,
                  xs, sqc, sqr, m1r, m2r, m1c, m2c, acc):
    r = pl.program_id(0)
    p = pl.program_id(1)

    @pl.when((r == 0) & (p == 0))
    def _():
        _prep(f_ref, xs, sqc, sqr)

    @pl.when(p == 0)
    def _():
        m1r[...] = jnp.full((B, P), jnp.inf, jnp.float32)
        m2r[...] = jnp.full((B, P), jnp.inf, jnp.float32)
        m1c[...] = jnp.full((P, B), jnp.inf, jnp.float32)
        m2c[...] = jnp.full((P, B), jnp.inf, jnp.float32)

    i = ii_ref[p]
    j = jj_ref[p]
    xq = xs[r, pl.ds(i * P, P)]
    xk = xs[r, pl.ds(j * P, P)]
    sqq = sqr[r, pl.ds(i, 1)]
    sqk = sqc[r, pl.ds(j * P, P)]
    g = jax.lax.dot_general(xk, xq, (((1,), (1,)), ((), ())),
                            preferred_element_type=jnp.float32)
    d2 = (sqk - 2.0 * g) + sqq
    dq = jnp.min(d2, axis=0, keepdims=True)
    dk = jnp.min(d2, axis=1, keepdims=True)

    rows = jax.lax.broadcasted_iota(jnp.int32, (B, P), 0)
    urow = rows == i
    dqb = jnp.broadcast_to(dq, (B, P))
    o1 = m1r[...]
    m1r[...] = jnp.where(urow, jnp.minimum(o1, dqb), o1)
    m2r[...] = jnp.where(urow & (dqb < o1), o1,
                         jnp.where(urow, jnp.minimum(m2r[...], dqb), m2r[...]))

    cols = jax.lax.broadcasted_iota(jnp.int32, (P, B), 1)
    ucol = cols == j
    dkb = jnp.broadcast_to(dk, (P, B))
    c1 = m1c[...]
    m1c[...] = jnp.where(ucol, jnp.minimum(c1, dkb), c1)
    m2c[...] = jnp.where(ucol & (dkb < c1), c1,
                         jnp.where(ucol, jnp.minimum(m2c[...], dkb), m2c[...]))

    @pl.when(p == NPAIR - 1)
    def _():
        t1 = m1c[...].T
        t2 = m2c[...].T
        f1, f2 = _merge_top2(m1r[...], m2r[...], t1, t2)
        contrib = 0.5 * (jnp.sqrt(jnp.maximum(f1, 1e-12)) +
                         jnp.sqrt(jnp.maximum(f2, 1e-12)))

        @pl.when(r == 0)
        def _():
            acc[...] = 0.5 * contrib

        @pl.when(r == 1)
        def _():
            tot = acc[...] + 0.5 * contrib
            ps_ref[...] = tot
            sc_ref[...] = jnp.broadcast_to(jnp.max(tot, axis=1, keepdims=True),
                                           (B, 128))


def kernel(features):
    pairs = [(a, b) for a in range(B) for b in range(a + 1, B)]
    ii = jnp.asarray(np.array([a for a, _ in pairs], dtype=np.int32))
    jj = jnp.asarray(np.array([b for _, b in pairs], dtype=np.int32))

    ps, sc = pl.pallas_call(
        _fused_kernel,
        grid_spec=pltpu.PrefetchScalarGridSpec(
            num_scalar_prefetch=2,
            grid=(2, NPAIR),
            in_specs=[pl.BlockSpec((B, P, D), lambda r, p, ii, jj: (0, 0, 0))],
            out_specs=[
                pl.BlockSpec((B, P), lambda r, p, ii, jj: (0, 0)),
                pl.BlockSpec((B, 128), lambda r, p, ii, jj: (0, 0)),
            ],
            scratch_shapes=[
                pltpu.VMEM((2, N, D), jnp.bfloat16),
                pltpu.VMEM((2, N, 1), jnp.float32),
                pltpu.VMEM((2, B, P), jnp.float32),
                pltpu.VMEM((B, P), jnp.float32),
                pltpu.VMEM((B, P), jnp.float32),
                pltpu.VMEM((P, B), jnp.float32),
                pltpu.VMEM((P, B), jnp.float32),
                pltpu.VMEM((B, P), jnp.float32),
            ],
        ),
        out_shape=[
            jax.ShapeDtypeStruct((B, P), jnp.float32),
            jax.ShapeDtypeStruct((B, 128), jnp.float32),
        ],
    )(ii, jj, features)

    scores = sc[:, 0]
    scores_pixel = ps.reshape(B, PH, PW)
    return scores, scores_pixel
